# baseline (speedup 1.0000x reference)
import functools

import jax
import jax.numpy as jnp
from jax import lax
from jax.experimental import pallas as pl
from jax.experimental.pallas import tpu as pltpu
from jax.experimental.pallas import tpu_sc as plsc

NC = 2
NS = 16
NW = NC * NS

IW = 128
K = 8


@jax.jit
def _embedding_gather(idx2d, table):
    n_rows, _ = idx2d.shape
    V, D = table.shape
    rows_per_w = n_rows // NW
    n_groups = rows_per_w // K
    n_idx = n_rows * IW

    mesh = plsc.VectorSubcoreMesh(
        core_axis_name="c", subcore_axis_name="s",
        num_cores=NC, num_subcores=NS,
    )

    @functools.partial(
        pl.kernel,
        out_type=jax.ShapeDtypeStruct((n_idx, D), jnp.float32),
        mesh=mesh,
        scratch_types=[
            pltpu.VMEM((K, IW), jnp.int32),
            pltpu.VMEM((K * IW, D), jnp.float32),
            pltpu.SemaphoreType.DMA,
        ],
        compiler_params=pltpu.CompilerParams(use_tc_tiling_on_sc=False),
    )
    def body(idx_hbm, table_hbm, out_hbm, idx_v, rows_v, sem):
        wid = lax.axis_index("s") * NC + lax.axis_index("c")
        row0 = wid * rows_per_w

        def group(g, carry):
            r = row0 + g * K
            pltpu.sync_copy(idx_hbm.at[pl.ds(r, K)], idx_v)
            for j in range(K):
                pltpu.async_copy(
                    table_hbm.at[idx_v.at[j]],
                    rows_v.at[pl.ds(j * IW, IW)],
                    sem,
                )
            for j in range(K):
                pltpu.make_async_copy(
                    table_hbm.at[idx_v.at[j]],
                    rows_v.at[pl.ds(j * IW, IW)],
                    sem,
                ).wait()
            pltpu.sync_copy(rows_v, out_hbm.at[pl.ds(r * IW, K * IW)])
            return carry

        lax.fori_loop(0, n_groups, group, 0)

    return body(idx2d, table)


def kernel(words, table):
    B, H = words.shape
    _, D = table.shape
    idx2d = words.astype(jnp.int32).reshape((B * H) // IW, IW)
    out = _embedding_gather(idx2d, table)
    return out.reshape(B, H, D)


# speedup vs baseline: 1.0080x; 1.0080x over previous
import functools

import jax
import jax.numpy as jnp
from jax import lax
from jax.experimental import pallas as pl
from jax.experimental.pallas import tpu as pltpu
from jax.experimental.pallas import tpu_sc as plsc

NC = 2
NS = 16
NW = NC * NS

IW = 128
KB = 8
KH = KB // 2


@jax.jit
def _embedding_gather(idx2d, table):
    n_rows, _ = idx2d.shape
    V, D = table.shape
    rows_per_w = n_rows // NW
    n_blocks = rows_per_w // KB
    n_idx = n_rows * IW

    mesh = plsc.VectorSubcoreMesh(
        core_axis_name="c", subcore_axis_name="s",
        num_cores=NC, num_subcores=NS,
    )

    @functools.partial(
        pl.kernel,
        out_type=jax.ShapeDtypeStruct((n_idx, D), jnp.float32),
        mesh=mesh,
        scratch_types=[
            pltpu.VMEM((KB, IW), jnp.int32),
            pltpu.VMEM((KH * IW, D), jnp.float32),
            pltpu.VMEM((KH * IW, D), jnp.float32),
            pltpu.SemaphoreType.DMA,
            pltpu.SemaphoreType.DMA,
        ],
        compiler_params=pltpu.CompilerParams(use_tc_tiling_on_sc=False),
    )
    def body(idx_hbm, table_hbm, out_hbm, idx_v, rows0, rows1, sem_g, sem_o):
        wid = lax.axis_index("s") * NC + lax.axis_index("c")
        row0 = wid * rows_per_w
        rows_bufs = (rows0, rows1)

        def fire(buf, jofs):
            for j in range(KH):
                pltpu.async_copy(
                    table_hbm.at[idx_v.at[jofs + j]],
                    rows_bufs[buf].at[pl.ds(j * IW, IW)],
                    sem_g,
                )

        def drain(buf, jofs):
            for j in range(KH):
                pltpu.make_async_copy(
                    table_hbm.at[idx_v.at[jofs + j]],
                    rows_bufs[buf].at[pl.ds(j * IW, IW)],
                    sem_g,
                ).wait()

        def store(h, buf, half):
            pltpu.async_copy(
                rows_bufs[buf],
                out_hbm.at[pl.ds((row0 + h * KB + half * KH) * IW, KH * IW)],
                sem_o,
            )

        def store_wait(h, buf, half):
            pltpu.make_async_copy(
                rows_bufs[buf],
                out_hbm.at[pl.ds((row0 + h * KB + half * KH) * IW, KH * IW)],
                sem_o,
            ).wait()

        def block(h, carry):
            pltpu.sync_copy(idx_hbm.at[pl.ds(row0 + h * KB, KB)], idx_v)
            fire(0, 0)

            @pl.when(h > 0)
            def _():
                store_wait(h - 1, 1, 1)
            drain(0, 0)
            store(h, 0, 0)
            fire(1, KH)
            store_wait(h, 0, 0)
            drain(1, KH)
            store(h, 1, 1)
            return carry

        lax.fori_loop(0, n_blocks, block, 0)
        store_wait(n_blocks - 1, 1, 1)

    return body(idx2d, table)


def kernel(words, table):
    B, H = words.shape
    _, D = table.shape
    idx2d = words.astype(jnp.int32).reshape((B * H) // IW, IW)
    out = _embedding_gather(idx2d, table)
    return out.reshape(B, H, D)


# speedup vs baseline: 1.0632x; 1.0548x over previous
import functools

import jax
import jax.numpy as jnp
from jax import lax
from jax.experimental import pallas as pl
from jax.experimental.pallas import tpu as pltpu
from jax.experimental.pallas import tpu_sc as plsc

NC = 2
NS = 16
NW = NC * NS

IW = 128
KB = 8
KH = KB // 2


@jax.jit
def _embedding_gather(idx2d, table2):
    n_rows, _ = idx2d.shape
    V2, D = table2.shape
    rows_per_w = n_rows // NW
    n_blocks = rows_per_w // KB
    n_idx = n_rows * IW

    mesh = plsc.VectorSubcoreMesh(
        core_axis_name="c", subcore_axis_name="s",
        num_cores=NC, num_subcores=NS,
    )

    @functools.partial(
        pl.kernel,
        out_type=jax.ShapeDtypeStruct((n_idx, D), jnp.float32),
        mesh=mesh,
        scratch_types=[
            pltpu.VMEM((KB, IW), jnp.int32),
            pltpu.VMEM((KH * IW, D), jnp.float32),
            pltpu.VMEM((KH * IW, D), jnp.float32),
            pltpu.SemaphoreType.DMA,
            pltpu.SemaphoreType.DMA,
        ],
        compiler_params=pltpu.CompilerParams(use_tc_tiling_on_sc=False),
    )
    def body(idx_hbm, table_hbm, out_hbm, idx_v, rows0, rows1, sem_g, sem_o):
        wid = lax.axis_index("s") * NC + lax.axis_index("c")
        row0 = wid * rows_per_w
        rows_bufs = (rows0, rows1)

        def fire(buf, jofs):
            for j in range(KH):
                pltpu.async_copy(
                    table_hbm.at[idx_v.at[jofs + j]],
                    rows_bufs[buf].at[pl.ds(j * IW, IW)],
                    sem_g,
                )

        def drain(buf, jofs):
            for j in range(KH):
                pltpu.make_async_copy(
                    table_hbm.at[idx_v.at[jofs + j]],
                    rows_bufs[buf].at[pl.ds(j * IW, IW)],
                    sem_g,
                ).wait()

        def store(h, buf, half):
            pltpu.async_copy(
                rows_bufs[buf],
                out_hbm.at[pl.ds((row0 + h * KB + half * KH) * IW, KH * IW)],
                sem_o,
            )

        def store_wait(h, buf, half):
            pltpu.make_async_copy(
                rows_bufs[buf],
                out_hbm.at[pl.ds((row0 + h * KB + half * KH) * IW, KH * IW)],
                sem_o,
            ).wait()

        def block(h, carry):
            pltpu.sync_copy(idx_hbm.at[pl.ds(row0 + h * KB, KB)], idx_v)
            fire(0, 0)

            @pl.when(h > 0)
            def _():
                store_wait(h - 1, 1, 1)
            drain(0, 0)
            store(h, 0, 0)
            fire(1, KH)
            store_wait(h, 0, 0)
            drain(1, KH)
            store(h, 1, 1)
            return carry

        lax.fori_loop(0, n_blocks, block, 0)
        store_wait(n_blocks - 1, 1, 1)

    return body(idx2d, table2)


def kernel(words, table):
    B, H = words.shape
    V, D = table.shape
    idx2d = (words.astype(jnp.int32) * 2).reshape((B * H) // IW, IW)
    table2 = jnp.pad(table, ((0, 0), (0, D))).reshape(2 * V, D)
    out = _embedding_gather(idx2d, table2)
    return out.reshape(B, H, D)


# speedup vs baseline: 1.4415x; 1.3557x over previous
import functools

import jax
import jax.numpy as jnp
from jax import lax
from jax.experimental import pallas as pl
from jax.experimental.pallas import tpu as pltpu
from jax.experimental.pallas import tpu_sc as plsc

NC = 2
NS = 16
NW = NC * NS

IW = 128
KB = 8
KH = KB // 2


@jax.jit
def _embedding_gather(idx2d, table2):
    n_rows, _ = idx2d.shape
    V2, D = table2.shape
    rows_per_w = n_rows // NW
    n_blocks = rows_per_w // KB
    n_idx = n_rows * IW

    mesh = plsc.VectorSubcoreMesh(
        core_axis_name="c", subcore_axis_name="s",
        num_cores=NC, num_subcores=NS,
    )

    @functools.partial(
        pl.kernel,
        out_type=jax.ShapeDtypeStruct((n_idx, 2 * D), jnp.float32),
        mesh=mesh,
        scratch_types=[
            pltpu.VMEM((KB, IW), jnp.int32),
            pltpu.VMEM((KH * IW, D), jnp.float32),
            pltpu.VMEM((KH * IW, D), jnp.float32),
            pltpu.SemaphoreType.DMA,
            pltpu.SemaphoreType.DMA,
        ],
        compiler_params=pltpu.CompilerParams(use_tc_tiling_on_sc=False),
    )
    def body(idx_hbm, table_hbm, out_hbm, idx_v, rows0, rows1, sem_g, sem_o):
        wid = lax.axis_index("s") * NC + lax.axis_index("c")
        row0 = wid * rows_per_w
        rows_bufs = (rows0, rows1)

        def fire(buf, jofs):
            for j in range(KH):
                pltpu.async_copy(
                    table_hbm.at[idx_v.at[jofs + j]],
                    rows_bufs[buf].at[pl.ds(j * IW, IW)],
                    sem_g,
                )

        def drain(buf, jofs):
            for j in range(KH):
                pltpu.make_async_copy(
                    table_hbm.at[idx_v.at[jofs + j]],
                    rows_bufs[buf].at[pl.ds(j * IW, IW)],
                    sem_g,
                ).wait()

        def store(h, buf, half):
            pltpu.async_copy(
                rows_bufs[buf],
                out_hbm.at[pl.ds((row0 + h * KB + half * KH) * IW, KH * IW),
                           pl.ds(0, D)],
                sem_o,
            )

        def store_wait(h, buf, half):
            pltpu.make_async_copy(
                rows_bufs[buf],
                out_hbm.at[pl.ds((row0 + h * KB + half * KH) * IW, KH * IW),
                           pl.ds(0, D)],
                sem_o,
            ).wait()

        def block(h, carry):
            pltpu.sync_copy(idx_hbm.at[pl.ds(row0 + h * KB, KB)], idx_v)
            fire(0, 0)

            @pl.when(h > 0)
            def _():
                store_wait(h - 1, 1, 1)
            drain(0, 0)
            store(h, 0, 0)
            fire(1, KH)
            store_wait(h, 0, 0)
            drain(1, KH)
            store(h, 1, 1)
            return carry

        lax.fori_loop(0, n_blocks, block, 0)
        store_wait(n_blocks - 1, 1, 1)

    return body(idx2d, table2)


def kernel(words, table):
    B, H = words.shape
    V, D = table.shape
    idx2d = (words.astype(jnp.int32) * 2).reshape((B * H) // IW, IW)
    table2 = jnp.pad(table, ((0, 0), (0, D))).reshape(2 * V, D)
    out = _embedding_gather(idx2d, table2)
    return out[:, :D].reshape(B, H, D)
